# Initial kernel scaffold; baseline (speedup 1.0000x reference)
import jax
import jax.numpy as jnp
from jax.experimental import pallas as pl


def kernel(x, pos, batch):
    raise NotImplementedError("write your pallas kernel here")


# speedup vs baseline: 1.9571x; 1.9571x over previous
import functools

import jax
import jax.numpy as jnp
from jax.experimental import pallas as pl
from jax.experimental.pallas import tpu as pltpu

_RADII = [0.2, 0.4]
_MAX_POINTS = [32, 64]

_N = 8192
_T = 512
_G = _N // _T

_R2_0 = _RADII[0] * _RADII[0]
_R2_1 = _RADII[1] * _RADII[1]


def _count_kernel(seg_ref, pos_q_ref, pos_k_ref, b_q_ref, b_k_ref, out_ref):
    i = pl.program_id(0)
    j = pl.program_id(1)

    @pl.when(j == 0)
    def _init():
        out_ref[...] = jnp.zeros_like(out_ref)

    lo_q = seg_ref[i]
    hi_q = seg_ref[_G + i]
    lo_k = seg_ref[j]
    hi_k = seg_ref[_G + j]
    overlap = jnp.logical_and(lo_q <= hi_k, lo_k <= hi_q)

    @pl.when(overlap)
    def _compute():
        pq = pos_q_ref[...]
        pk = pos_k_ref[...]
        dx = pk[0][:, None] - pq[0][None, :]
        dy = pk[1][:, None] - pq[1][None, :]
        dz = pk[2][:, None] - pq[2][None, :]
        d2 = dx * dx + dy * dy + dz * dz
        same = b_k_ref[0][:, None] == b_q_ref[0][None, :]
        w0 = jnp.where((d2 <= _R2_0) & same, 1.0, 0.0)
        w1 = jnp.where((d2 <= _R2_1) & same, 1.0, 0.0)
        out_ref[0:1, :] += jnp.sum(w0, axis=0, keepdims=True)
        out_ref[1:2, :] += jnp.sum(w1, axis=0, keepdims=True)


@jax.jit
def kernel(x, pos, batch):
    pos_t = pos.T
    batch2d = batch.reshape(1, _N).astype(jnp.int32)

    btiles = batch2d.reshape(_G, _T)
    seg = jnp.concatenate([btiles[:, 0], btiles[:, -1]]).astype(jnp.int32)

    grid_spec = pltpu.PrefetchScalarGridSpec(
        num_scalar_prefetch=1,
        grid=(_G, _G),
        in_specs=[
            pl.BlockSpec((3, _T), lambda i, j, seg: (0, i)),
            pl.BlockSpec((3, _T), lambda i, j, seg: (0, j)),
            pl.BlockSpec((1, _T), lambda i, j, seg: (0, i)),
            pl.BlockSpec((1, _T), lambda i, j, seg: (0, j)),
        ],
        out_specs=pl.BlockSpec((2, _T), lambda i, j, seg: (0, i)),
    )

    counts = pl.pallas_call(
        _count_kernel,
        grid_spec=grid_spec,
        out_shape=jax.ShapeDtypeStruct((2, _N), jnp.float32),
    )(seg, pos_t, pos_t, batch2d, batch2d)

    cnt0 = jnp.minimum(counts[0], float(_MAX_POINTS[0])) / float(_MAX_POINTS[0])
    cnt1 = jnp.minimum(counts[1], float(_MAX_POINTS[1])) / float(_MAX_POINTS[1])
    feats = jnp.concatenate(
        [x, pos, cnt0[:, None], cnt1[:, None]], axis=1
    )
    return (feats, pos, batch)
